# baseline (speedup 1.0000x reference)
import jax
import jax.numpy as jnp
from jax.experimental import pallas as pl
from jax.experimental.pallas import tpu as pltpu


def _round_up(n, m):
    return ((n + m - 1) // m) * m


_CH = 1024


def _make_stream_kernel(half, nch):
    ch = half // nch

    def body(x_hbm, w_ref, b_ref, o_hbm, x_buf, o_buf, in_sem, out_sem):
        row0 = pl.program_id(0) * half

        def dma_in(step):
            return pltpu.make_async_copy(
                x_hbm.at[pl.ds(row0 + step * ch, ch), :],
                x_buf.at[step], in_sem.at[step])

        for s in range(nch):
            dma_in(s).start()

        wb = w_ref[...].astype(jnp.bfloat16)
        brow = b_ref[...]

        for step in range(nch):
            dma_in(step).wait()
            xb = x_buf[step].astype(jnp.bfloat16)
            acc = jax.lax.dot_general(
                xb, wb, (((1,), (1,)), ((), ())),
                preferred_element_type=jnp.float32)
            o_buf[step * ch:(step + 1) * ch, :] = acc + brow

        out_dma = pltpu.make_async_copy(
            o_buf, o_hbm.at[pl.ds(row0, half), :], out_sem)
        out_dma.start()
        out_dma.wait()

    return body


def _stream_call(x_p, w_p, b_row, B_pad, CPAD, D, cost):
    half = B_pad // 2
    nch = 2 if half % 2 == 0 else 1
    return pl.pallas_call(
        _make_stream_kernel(half, nch),
        out_shape=jax.ShapeDtypeStruct((B_pad, CPAD), jnp.float32),
        grid_spec=pltpu.PrefetchScalarGridSpec(
            num_scalar_prefetch=0,
            grid=(2,),
            in_specs=[
                pl.BlockSpec(memory_space=pl.ANY),
                pl.BlockSpec((CPAD, D), lambda i: (0, 0)),
                pl.BlockSpec((1, CPAD), lambda i: (0, 0)),
            ],
            out_specs=pl.BlockSpec(memory_space=pl.ANY),
            scratch_shapes=[
                pltpu.VMEM((nch, half // nch, D), jnp.float32),
                pltpu.VMEM((half, CPAD), jnp.float32),
                pltpu.SemaphoreType.DMA((nch,)),
                pltpu.SemaphoreType.DMA,
            ],
        ),
        compiler_params=pltpu.CompilerParams(
            dimension_semantics=("parallel",),
            vmem_limit_bytes=56 * 1024 * 1024),
        cost_estimate=cost,
    )(x_p, w_p, b_row)


def _emitter_kernel(x_ref, w_ref, b_ref, o_ref):
    xb = x_ref[...].astype(jnp.bfloat16)
    wb = w_ref[...].astype(jnp.bfloat16)
    acc = jax.lax.dot_general(
        xb, wb, (((1,), (1,)), ((), ())),
        preferred_element_type=jnp.float32)
    o_ref[...] = acc + b_ref[...]


def _emitter_call(x_p, w_p, b_row, B_pad, TB, CPAD, D, cost):
    return pl.pallas_call(
        _emitter_kernel,
        out_shape=jax.ShapeDtypeStruct((B_pad, CPAD), jnp.float32),
        grid_spec=pltpu.PrefetchScalarGridSpec(
            num_scalar_prefetch=0,
            grid=(B_pad // TB,),
            in_specs=[
                pl.BlockSpec((TB, D), lambda i: (i, 0)),
                pl.BlockSpec((CPAD, D), lambda i: (0, 0)),
                pl.BlockSpec((1, CPAD), lambda i: (0, 0)),
            ],
            out_specs=pl.BlockSpec((TB, CPAD), lambda i: (i, 0)),
        ),
        compiler_params=pltpu.CompilerParams(
            dimension_semantics=("parallel",),
            vmem_limit_bytes=56 * 1024 * 1024),
        cost_estimate=cost,
    )(x_p, w_p, b_row)


def kernel(x, weight, bias):
    B, D = x.shape
    C, D2 = weight.shape
    assert D == D2 and bias.shape == (C,)

    CPAD = _round_up(C, 128)
    B_pad = _round_up(B, 2 * _CH)
    half = B_pad // 2

    x = x.astype(jnp.float32)
    x_p = x if B_pad == B else jnp.pad(x, ((0, B_pad - B), (0, 0)))
    w_p = weight.astype(jnp.float32)
    if CPAD != C:
        w_p = jnp.pad(w_p, ((0, CPAD - C), (0, 0)))
    b_row = jnp.pad(bias.astype(jnp.float32), (0, CPAD - C)).reshape(1, CPAD)

    cost = pl.CostEstimate(
        flops=2 * B * D * C,
        transcendentals=0,
        bytes_accessed=int(B_pad * D * 4 + D * CPAD * 4
                           + CPAD * 4 + B_pad * CPAD * 4),
    )

    vmem_need = half * D * 4 + half * CPAD * 4 + CPAD * D * 4
    if vmem_need <= 40 * 1024 * 1024:
        out_padded = _stream_call(x_p, w_p, b_row, B_pad, CPAD, D, cost)
    else:
        TB = 2048 if B_pad % 2048 == 0 else _CH
        out_padded = _emitter_call(x_p, w_p, b_row, B_pad, TB, CPAD, D, cost)

    return out_padded[:B, :C]


# speedup vs baseline: 1.3472x; 1.3472x over previous
import jax
import jax.numpy as jnp
from jax.experimental import pallas as pl
from jax.experimental.pallas import tpu as pltpu


def _round_up(n, m):
    return ((n + m - 1) // m) * m


def _linear_kernel(x_ref, w_ref, b_ref, o_ref):
    acc = jax.lax.dot_general(
        x_ref[...], w_ref[...], (((1,), (1,)), ((), ())),
        preferred_element_type=jnp.float32)
    o_ref[...] = acc + b_ref[...]


def kernel(x, weight, bias):
    B, D = x.shape
    C, D2 = weight.shape
    assert D == D2 and bias.shape == (C,)

    CPAD = _round_up(C, 128)

    TB = _round_up(B, 8)
    while TB > 8 and (2 * TB * (D + CPAD) * 4 + 2 * CPAD * D * 4
                      > 48 * 1024 * 1024 or TB * 2 > _round_up(B, 8)):
        TB = _round_up(TB // 2, 8)
    B_pad = _round_up(B, TB)

    x = x.astype(jnp.float32)
    x_p = x if B_pad == B else jnp.pad(x, ((0, B_pad - B), (0, 0)))
    w_p = weight.astype(jnp.float32)
    if CPAD != C:
        w_p = jnp.pad(w_p, ((0, CPAD - C), (0, 0)))
    b_row = jnp.pad(bias.astype(jnp.float32), (0, CPAD - C)).reshape(1, CPAD)

    cost = pl.CostEstimate(
        flops=2 * B * D * C,
        transcendentals=0,
        bytes_accessed=int(B_pad * D * 4 + D * CPAD * 4
                           + CPAD * 4 + B_pad * CPAD * 4),
    )

    out_padded = pl.pallas_call(
        _linear_kernel,
        out_shape=jax.ShapeDtypeStruct((B_pad, CPAD), jnp.float32),
        grid_spec=pltpu.PrefetchScalarGridSpec(
            num_scalar_prefetch=0,
            grid=(B_pad // TB,),
            in_specs=[
                pl.BlockSpec((TB, D), lambda i: (i, 0)),
                pl.BlockSpec((CPAD, D), lambda i: (0, 0)),
                pl.BlockSpec((1, CPAD), lambda i: (0, 0)),
            ],
            out_specs=pl.BlockSpec((TB, CPAD), lambda i: (i, 0)),
        ),
        compiler_params=pltpu.CompilerParams(
            dimension_semantics=("parallel",),
            vmem_limit_bytes=56 * 1024 * 1024),
        cost_estimate=cost,
    )(x_p, w_p, b_row)

    return out_padded[:B, :C]
